# Initial kernel scaffold; baseline (speedup 1.0000x reference)
import jax
import jax.numpy as jnp
from jax.experimental import pallas as pl


def kernel(x, segment_ids, gate_W, gate_b, W1, b1, W2, b2):
    raise NotImplementedError("write your pallas kernel here")


# speedup vs baseline: 6.0677x; 6.0677x over previous
import functools

import jax
import jax.numpy as jnp
from jax.experimental import pallas as pl
from jax.experimental.pallas import tpu as pltpu

N = 100000
D = 128
G = 16
TOPK = 64
BLK = 2000
NBLK = N // BLK

PAD_ROWS = 784
PAD = PAD_ROWS * 128


def _pass1_body(seg_ref, x_ref, gwt_ref, gb_ref, w1_ref, b1_ref, w2_ref,
                b2_ref, gate_out_ref, out_ref, m_out_ref, l_out_ref,
                m_s, l_s, acc_s):
    i = pl.program_id(0)

    @pl.when(i == 0)
    def _init():
        m_s[...] = jnp.full((G, 1), -jnp.inf, jnp.float32)
        l_s[...] = jnp.zeros((G, 1), jnp.float32)
        acc_s[...] = jnp.zeros((G, D), jnp.float32)

    x = x_ref[...]
    gwt = gwt_ref[...]
    gate = jax.lax.dot_general(
        gwt, x, (((1,), (1,)), ((), ())),
        preferred_element_type=jnp.float32) + gb_ref[0, 0]
    gate_out_ref[...] = gate[None]

    seg = seg_ref[0]
    sids = jax.lax.broadcasted_iota(jnp.int32, (G, 1), 0)
    onehot = seg == sids

    bm = jnp.max(jnp.where(onehot, gate, -jnp.inf), axis=1, keepdims=True)
    m_old = m_s[...]
    m_new = jnp.maximum(m_old, bm)
    scale = jnp.where(m_new == m_old, 1.0, jnp.exp(m_old - m_new))
    m_row = jnp.sum(jnp.where(onehot, m_new, 0.0), axis=0, keepdims=True)
    p = jnp.exp(gate - m_row)
    w = jnp.where(onehot, p, 0.0)
    l_s[...] = l_s[...] * scale + jnp.sum(w, axis=1, keepdims=True)
    acc_s[...] = acc_s[...] * scale + jax.lax.dot_general(
        w, x, (((1,), (0,)), ((), ())), preferred_element_type=jnp.float32, precision=jax.lax.Precision.HIGHEST)
    m_s[...] = m_new

    @pl.when(i == NBLK - 1)
    def _fin():
        l = l_s[...]
        graph_x = acc_s[...] / jnp.where(l > 0, l, 1.0)
        h = jax.lax.dot_general(
            graph_x, w1_ref[...], (((1,), (0,)), ((), ())),
            preferred_element_type=jnp.float32) + b1_ref[...]
        h = jax.nn.gelu(h)
        o = jax.lax.dot_general(
            h, w2_ref[...], (((1,), (0,)), ((), ())),
            preferred_element_type=jnp.float32) + b2_ref[0, 0]
        out_ref[...] = o
        m_out_ref[...] = m_s[...]
        l_out_ref[...] = l


def _topk_body(gate_ref, seg_ref, m_ref, l_ref, vals_ref, idx_ref, v_s):
    m = m_ref[...]
    l = l_ref[...]
    c = m + jnp.log(jnp.where(l > 0, l, 1.0))
    g = gate_ref[...]
    seg = seg_ref[...]
    adj = jnp.full_like(g, -jnp.inf)
    for s in range(G):
        adj = jnp.where(seg == s, g - c[s, 0], adj)
    v_s[...] = adj

    ri = jax.lax.broadcasted_iota(jnp.int32, (PAD_ROWS, 128), 0)
    ci = jax.lax.broadcasted_iota(jnp.int32, (PAD_ROWS, 128), 1)
    flat = ri * 128 + ci

    def body(k, _):
        v = v_s[...]
        mx = jnp.max(v)
        cand = jnp.where(v == mx, flat, jnp.int32(2**30))
        idx = jnp.min(cand)
        vals_ref[pl.ds(k, 1), :] = jnp.exp(mx).reshape(1, 1)
        idx_ref[pl.ds(k, 1), :] = idx.reshape(1, 1)
        v_s[...] = jnp.where(cand == idx, -jnp.inf, v)
        return 0

    jax.lax.fori_loop(0, TOPK, body, 0)


@functools.partial(jax.jit, static_argnames=("interpret",))
def kernel(x, segment_ids, gate_W, gate_b, W1, b1, W2, b2, interpret=False):
    seg3 = segment_ids.reshape(NBLK, 1, BLK)
    gwt = gate_W.reshape(1, D)
    gb = gate_b.reshape(1, 1)
    b1r = b1.reshape(1, D)
    b2r = b2.reshape(1, 1)

    gate_out, out, m, l = pl.pallas_call(
        _pass1_body,
        grid=(NBLK,),
        in_specs=[
            pl.BlockSpec((1, 1, BLK), lambda i: (i, 0, 0)),
            pl.BlockSpec((BLK, D), lambda i: (i, 0)),
            pl.BlockSpec((1, D), lambda i: (0, 0)),
            pl.BlockSpec((1, 1), lambda i: (0, 0)),
            pl.BlockSpec((D, D), lambda i: (0, 0)),
            pl.BlockSpec((1, D), lambda i: (0, 0)),
            pl.BlockSpec((D, 1), lambda i: (0, 0)),
            pl.BlockSpec((1, 1), lambda i: (0, 0)),
        ],
        out_specs=[
            pl.BlockSpec((1, 1, BLK), lambda i: (i, 0, 0)),
            pl.BlockSpec((G, 1), lambda i: (0, 0)),
            pl.BlockSpec((G, 1), lambda i: (0, 0)),
            pl.BlockSpec((G, 1), lambda i: (0, 0)),
        ],
        out_shape=[
            jax.ShapeDtypeStruct((NBLK, 1, BLK), jnp.float32),
            jax.ShapeDtypeStruct((G, 1), jnp.float32),
            jax.ShapeDtypeStruct((G, 1), jnp.float32),
            jax.ShapeDtypeStruct((G, 1), jnp.float32),
        ],
        scratch_shapes=[
            pltpu.VMEM((G, 1), jnp.float32),
            pltpu.VMEM((G, 1), jnp.float32),
            pltpu.VMEM((G, D), jnp.float32),
        ],
        interpret=interpret,
    )(seg3, x, gwt, gb, W1, b1r, W2, b2r)

    gate_pad = jnp.pad(gate_out.reshape(N), (0, PAD - N),
                       constant_values=-jnp.inf).reshape(PAD_ROWS, 128)
    seg_pad = jnp.pad(segment_ids, (0, PAD - N),
                      constant_values=-1).reshape(PAD_ROWS, 128)

    vals, idx = pl.pallas_call(
        _topk_body,
        out_shape=[
            jax.ShapeDtypeStruct((TOPK, 1), jnp.float32),
            jax.ShapeDtypeStruct((TOPK, 1), jnp.int32),
        ],
        scratch_shapes=[pltpu.VMEM((PAD_ROWS, 128), jnp.float32)],
        interpret=interpret,
    )(gate_pad, seg_pad, m, l)

    return out, vals.reshape(TOPK), idx.reshape(TOPK)


# speedup vs baseline: 6.9651x; 1.1479x over previous
import functools

import jax
import jax.numpy as jnp
from jax.experimental import pallas as pl
from jax.experimental.pallas import tpu as pltpu

N = 100000
D = 128
G = 16
TOPK = 64
BLK = 5000
NBLK = N // BLK

PAD_ROWS = 784
PAD = PAD_ROWS * 128


def _pass1_body(seg_ref, x_ref, gwt_ref, gb_ref, w1_ref, b1_ref, w2_ref,
                b2_ref, gate_out_ref, out_ref, m_out_ref, l_out_ref,
                m_s, l_s, acc_s):
    i = pl.program_id(0)

    @pl.when(i == 0)
    def _init():
        m_s[...] = jnp.full((G, 1), -jnp.inf, jnp.float32)
        l_s[...] = jnp.zeros((G, 1), jnp.float32)
        acc_s[...] = jnp.zeros((G, D), jnp.float32)

    x = x_ref[...]
    gwt = gwt_ref[...]
    gate = jax.lax.dot_general(
        gwt, x, (((1,), (1,)), ((), ())),
        preferred_element_type=jnp.float32) + gb_ref[0, 0]
    gate_out_ref[...] = gate[None]

    seg = seg_ref[0]
    sids = jax.lax.broadcasted_iota(jnp.int32, (G, 1), 0)
    onehot = seg == sids

    bm = jnp.max(jnp.where(onehot, gate, -jnp.inf), axis=1, keepdims=True)
    m_old = m_s[...]
    m_new = jnp.maximum(m_old, bm)
    scale = jnp.where(m_new == m_old, 1.0, jnp.exp(m_old - m_new))
    m_row = jnp.sum(jnp.where(onehot, m_new, 0.0), axis=0, keepdims=True)
    p = jnp.exp(gate - m_row)
    w = jnp.where(onehot, p, 0.0)
    l_s[...] = l_s[...] * scale + jnp.sum(w, axis=1, keepdims=True)
    acc_s[...] = acc_s[...] * scale + jax.lax.dot_general(
        w, x, (((1,), (0,)), ((), ())), preferred_element_type=jnp.float32,
        precision=jax.lax.Precision.HIGHEST)
    m_s[...] = m_new

    @pl.when(i == NBLK - 1)
    def _fin():
        l = l_s[...]
        graph_x = acc_s[...] / jnp.where(l > 0, l, 1.0)
        h = jax.lax.dot_general(
            graph_x, w1_ref[...], (((1,), (0,)), ((), ())),
            preferred_element_type=jnp.float32) + b1_ref[...]
        h = jax.nn.gelu(h)
        o = jax.lax.dot_general(
            h, w2_ref[...], (((1,), (0,)), ((), ())),
            preferred_element_type=jnp.float32) + b2_ref[0, 0]
        out_ref[...] = o
        m_out_ref[...] = m_s[...]
        l_out_ref[...] = l


def _topk_body(gate_ref, seg_ref, m_ref, l_ref, vals_ref, idx_ref, v_s):
    m = m_ref[...]
    l = l_ref[...]
    c = m + jnp.log(jnp.where(l > 0, l, 1.0))
    g = gate_ref[...]
    seg = seg_ref[...]
    adj = jnp.full_like(g, -jnp.inf)
    for s in range(G):
        adj = jnp.where(seg == s, g - c[s, 0], adj)
    v_s[...] = adj

    ri = jax.lax.broadcasted_iota(jnp.int32, (PAD_ROWS, 128), 0)
    ci = jax.lax.broadcasted_iota(jnp.int32, (PAD_ROWS, 128), 1)
    flat = ri * 128 + ci

    def body(k, _):
        v = v_s[...]
        mx = jnp.max(v)
        cand = jnp.where(v == mx, flat, jnp.int32(2**30))
        idx = jnp.min(cand)
        vals_ref[pl.ds(k, 1), :] = jnp.exp(mx).reshape(1, 1)
        idx_ref[pl.ds(k, 1), :] = idx.reshape(1, 1)
        v_s[...] = jnp.where(cand == idx, -jnp.inf, v)
        return 0

    jax.lax.fori_loop(0, TOPK, body, 0)


@functools.partial(jax.jit, static_argnames=("interpret",))
def kernel(x, segment_ids, gate_W, gate_b, W1, b1, W2, b2, interpret=False):
    seg3 = segment_ids.reshape(NBLK, 1, BLK)
    gwt = gate_W.reshape(1, D)
    gb = gate_b.reshape(1, 1)
    b1r = b1.reshape(1, D)
    b2r = b2.reshape(1, 1)

    gate_out, out, m, l = pl.pallas_call(
        _pass1_body,
        grid=(NBLK,),
        in_specs=[
            pl.BlockSpec((1, 1, BLK), lambda i: (i, 0, 0)),
            pl.BlockSpec((BLK, D), lambda i: (i, 0)),
            pl.BlockSpec((1, D), lambda i: (0, 0)),
            pl.BlockSpec((1, 1), lambda i: (0, 0)),
            pl.BlockSpec((D, D), lambda i: (0, 0)),
            pl.BlockSpec((1, D), lambda i: (0, 0)),
            pl.BlockSpec((D, 1), lambda i: (0, 0)),
            pl.BlockSpec((1, 1), lambda i: (0, 0)),
        ],
        out_specs=[
            pl.BlockSpec((1, 1, BLK), lambda i: (i, 0, 0)),
            pl.BlockSpec((G, 1), lambda i: (0, 0)),
            pl.BlockSpec((G, 1), lambda i: (0, 0)),
            pl.BlockSpec((G, 1), lambda i: (0, 0)),
        ],
        out_shape=[
            jax.ShapeDtypeStruct((NBLK, 1, BLK), jnp.float32),
            jax.ShapeDtypeStruct((G, 1), jnp.float32),
            jax.ShapeDtypeStruct((G, 1), jnp.float32),
            jax.ShapeDtypeStruct((G, 1), jnp.float32),
        ],
        scratch_shapes=[
            pltpu.VMEM((G, 1), jnp.float32),
            pltpu.VMEM((G, 1), jnp.float32),
            pltpu.VMEM((G, D), jnp.float32),
        ],
        interpret=interpret,
    )(seg3, x, gwt, gb, W1, b1r, W2, b2r)

    gate_pad = jnp.pad(gate_out.reshape(N), (0, PAD - N),
                       constant_values=-jnp.inf).reshape(PAD_ROWS, 128)
    seg_pad = jnp.pad(segment_ids, (0, PAD - N),
                      constant_values=-1).reshape(PAD_ROWS, 128)

    vals, idx = pl.pallas_call(
        _topk_body,
        out_shape=[
            jax.ShapeDtypeStruct((TOPK, 1), jnp.float32),
            jax.ShapeDtypeStruct((TOPK, 1), jnp.int32),
        ],
        scratch_shapes=[pltpu.VMEM((PAD_ROWS, 128), jnp.float32)],
        interpret=interpret,
    )(gate_pad, seg_pad, m, l)

    return out, vals.reshape(TOPK), idx.reshape(TOPK)


# speedup vs baseline: 7.1050x; 1.0201x over previous
import functools

import jax
import jax.numpy as jnp
from jax.experimental import pallas as pl
from jax.experimental.pallas import tpu as pltpu

N = 100000
D = 128
G = 16
TOPK = 64
BLK = 5000
NBLK = N // BLK


def _body(seg_ref, x_ref, gwt_ref, gb_ref, w1_ref, b1_ref, w2_ref,
          b2_ref, out_ref, vals_ref, idx_ref,
          m_s, l_s, acc_s, gate_s, seg_s, v_s):
    i = pl.program_id(0)

    @pl.when(i == 0)
    def _init():
        m_s[...] = jnp.full((G, 1), -jnp.inf, jnp.float32)
        l_s[...] = jnp.zeros((G, 1), jnp.float32)
        acc_s[...] = jnp.zeros((G, D), jnp.float32)

    x = x_ref[...]
    gwt = gwt_ref[...]
    gate = jax.lax.dot_general(
        gwt, x, (((1,), (1,)), ((), ())),
        preferred_element_type=jnp.float32) + gb_ref[0, 0]
    seg = seg_ref[0]
    gate_s[pl.ds(i, 1), :] = gate
    seg_s[pl.ds(i, 1), :] = seg

    sids = jax.lax.broadcasted_iota(jnp.int32, (G, 1), 0)
    onehot = seg == sids

    bm = jnp.max(jnp.where(onehot, gate, -jnp.inf), axis=1, keepdims=True)
    m_old = m_s[...]
    m_new = jnp.maximum(m_old, bm)
    scale = jnp.where(m_new == m_old, 1.0, jnp.exp(m_old - m_new))
    m_row = jnp.sum(jnp.where(onehot, m_new, 0.0), axis=0, keepdims=True)
    p = jnp.exp(gate - m_row)
    w = jnp.where(onehot, p, 0.0)
    l_s[...] = l_s[...] * scale + jnp.sum(w, axis=1, keepdims=True)
    acc_s[...] = acc_s[...] * scale + jax.lax.dot_general(
        w, x, (((1,), (0,)), ((), ())), preferred_element_type=jnp.float32,
        precision=jax.lax.Precision.HIGHEST)
    m_s[...] = m_new

    @pl.when(i == NBLK - 1)
    def _fin():
        l = l_s[...]
        graph_x = acc_s[...] / jnp.where(l > 0, l, 1.0)
        h = jax.lax.dot_general(
            graph_x, w1_ref[...], (((1,), (0,)), ((), ())),
            preferred_element_type=jnp.float32) + b1_ref[...]
        h = jax.nn.gelu(h)
        o = jax.lax.dot_general(
            h, w2_ref[...], (((1,), (0,)), ((), ())),
            preferred_element_type=jnp.float32) + b2_ref[0, 0]
        out_ref[...] = o

        m = m_s[...]
        c = m + jnp.log(jnp.where(l > 0, l, 1.0))
        g = gate_s[...]
        sg = seg_s[...]
        adj = jnp.full_like(g, -jnp.inf)
        for s in range(G):
            adj = jnp.where(sg == s, g - c[s, 0], adj)
        v_s[...] = adj

        ri = jax.lax.broadcasted_iota(jnp.int32, (NBLK, BLK), 0)
        ci = jax.lax.broadcasted_iota(jnp.int32, (NBLK, BLK), 1)
        flat = ri * BLK + ci

        def body(k, _):
            v = v_s[...]
            mx = jnp.max(v)
            cand = jnp.where(v == mx, flat, jnp.int32(2**30))
            idx = jnp.min(cand)
            vals_ref[pl.ds(k, 1), :] = jnp.exp(mx).reshape(1, 1)
            idx_ref[pl.ds(k, 1), :] = idx.reshape(1, 1)
            v_s[...] = jnp.where(cand == idx, -jnp.inf, v)
            return 0

        jax.lax.fori_loop(0, TOPK, body, 0)


@functools.partial(jax.jit, static_argnames=("interpret",))
def kernel(x, segment_ids, gate_W, gate_b, W1, b1, W2, b2, interpret=False):
    seg3 = segment_ids.reshape(NBLK, 1, BLK)
    gwt = gate_W.reshape(1, D)
    gb = gate_b.reshape(1, 1)
    b1r = b1.reshape(1, D)
    b2r = b2.reshape(1, 1)

    out, vals, idx = pl.pallas_call(
        _body,
        grid=(NBLK,),
        in_specs=[
            pl.BlockSpec((1, 1, BLK), lambda i: (i, 0, 0)),
            pl.BlockSpec((BLK, D), lambda i: (i, 0)),
            pl.BlockSpec((1, D), lambda i: (0, 0)),
            pl.BlockSpec((1, 1), lambda i: (0, 0)),
            pl.BlockSpec((D, D), lambda i: (0, 0)),
            pl.BlockSpec((1, D), lambda i: (0, 0)),
            pl.BlockSpec((D, 1), lambda i: (0, 0)),
            pl.BlockSpec((1, 1), lambda i: (0, 0)),
        ],
        out_specs=[
            pl.BlockSpec((G, 1), lambda i: (0, 0)),
            pl.BlockSpec((TOPK, 1), lambda i: (0, 0)),
            pl.BlockSpec((TOPK, 1), lambda i: (0, 0)),
        ],
        out_shape=[
            jax.ShapeDtypeStruct((G, 1), jnp.float32),
            jax.ShapeDtypeStruct((TOPK, 1), jnp.float32),
            jax.ShapeDtypeStruct((TOPK, 1), jnp.int32),
        ],
        scratch_shapes=[
            pltpu.VMEM((G, 1), jnp.float32),
            pltpu.VMEM((G, 1), jnp.float32),
            pltpu.VMEM((G, D), jnp.float32),
            pltpu.VMEM((NBLK, BLK), jnp.float32),
            pltpu.VMEM((NBLK, BLK), jnp.int32),
            pltpu.VMEM((NBLK, BLK), jnp.float32),
        ],
        interpret=interpret,
    )(seg3, x, gwt, gb, W1, b1r, W2, b2r)

    return out, vals.reshape(TOPK), idx.reshape(TOPK)


# speedup vs baseline: 7.1703x; 1.0092x over previous
import functools

import jax
import jax.numpy as jnp
from jax.experimental import pallas as pl
from jax.experimental.pallas import tpu as pltpu

N = 100000
D = 128
G = 16
TOPK = 64
BLK = 10000
NBLK = N // BLK


def _body(seg_ref, x_ref, gwt_ref, gb_ref, w1_ref, b1_ref, w2_ref,
          b2_ref, out_ref, vals_ref, idx_ref,
          m_s, l_s, acc_s, gate_s, seg_s, v_s):
    i = pl.program_id(0)

    @pl.when(i == 0)
    def _init():
        m_s[...] = jnp.full((G, 1), -jnp.inf, jnp.float32)
        l_s[...] = jnp.zeros((G, 1), jnp.float32)
        acc_s[...] = jnp.zeros((G, D), jnp.float32)

    x = x_ref[...]
    gwt = gwt_ref[...]
    gate = jax.lax.dot_general(
        gwt, x, (((1,), (1,)), ((), ())),
        preferred_element_type=jnp.float32) + gb_ref[0, 0]
    seg = seg_ref[0]
    gate_s[pl.ds(i, 1), :] = gate
    seg_s[pl.ds(i, 1), :] = seg

    sids = jax.lax.broadcasted_iota(jnp.int32, (G, 1), 0)
    onehot = seg == sids

    bm = jnp.max(jnp.where(onehot, gate, -jnp.inf), axis=1, keepdims=True)
    m_old = m_s[...]
    m_new = jnp.maximum(m_old, bm)
    scale = jnp.where(m_new == m_old, 1.0, jnp.exp(m_old - m_new))
    m_row = jnp.sum(jnp.where(onehot, m_new, 0.0), axis=0, keepdims=True)
    p = jnp.exp(gate - m_row)
    w = jnp.where(onehot, p, 0.0)
    l_s[...] = l_s[...] * scale + jnp.sum(w, axis=1, keepdims=True)
    acc_s[...] = acc_s[...] * scale + jax.lax.dot_general(
        w, x, (((1,), (0,)), ((), ())), preferred_element_type=jnp.float32,
        precision=jax.lax.Precision.HIGHEST)
    m_s[...] = m_new

    @pl.when(i == NBLK - 1)
    def _fin():
        l = l_s[...]
        graph_x = acc_s[...] / jnp.where(l > 0, l, 1.0)
        h = jax.lax.dot_general(
            graph_x, w1_ref[...], (((1,), (0,)), ((), ())),
            preferred_element_type=jnp.float32) + b1_ref[...]
        h = jax.nn.gelu(h)
        o = jax.lax.dot_general(
            h, w2_ref[...], (((1,), (0,)), ((), ())),
            preferred_element_type=jnp.float32) + b2_ref[0, 0]
        out_ref[...] = o

        m = m_s[...]
        c = m + jnp.log(jnp.where(l > 0, l, 1.0))
        g = gate_s[...]
        sg = seg_s[...]
        adj = jnp.full_like(g, -jnp.inf)
        for s in range(G):
            adj = jnp.where(sg == s, g - c[s, 0], adj)
        v_s[...] = adj

        ri = jax.lax.broadcasted_iota(jnp.int32, (NBLK, BLK), 0)
        ci = jax.lax.broadcasted_iota(jnp.int32, (NBLK, BLK), 1)
        flat = ri * BLK + ci

        def body(k, _):
            v = v_s[...]
            mx = jnp.max(v)
            cand = jnp.where(v == mx, flat, jnp.int32(2**30))
            idx = jnp.min(cand)
            vals_ref[pl.ds(k, 1), :] = jnp.exp(mx).reshape(1, 1)
            idx_ref[pl.ds(k, 1), :] = idx.reshape(1, 1)
            v_s[...] = jnp.where(cand == idx, -jnp.inf, v)
            return 0

        jax.lax.fori_loop(0, TOPK, body, 0)


@functools.partial(jax.jit, static_argnames=("interpret",))
def kernel(x, segment_ids, gate_W, gate_b, W1, b1, W2, b2, interpret=False):
    seg3 = segment_ids.reshape(NBLK, 1, BLK)
    gwt = gate_W.reshape(1, D)
    gb = gate_b.reshape(1, 1)
    b1r = b1.reshape(1, D)
    b2r = b2.reshape(1, 1)

    out, vals, idx = pl.pallas_call(
        _body,
        grid=(NBLK,),
        in_specs=[
            pl.BlockSpec((1, 1, BLK), lambda i: (i, 0, 0)),
            pl.BlockSpec((BLK, D), lambda i: (i, 0)),
            pl.BlockSpec((1, D), lambda i: (0, 0)),
            pl.BlockSpec((1, 1), lambda i: (0, 0)),
            pl.BlockSpec((D, D), lambda i: (0, 0)),
            pl.BlockSpec((1, D), lambda i: (0, 0)),
            pl.BlockSpec((D, 1), lambda i: (0, 0)),
            pl.BlockSpec((1, 1), lambda i: (0, 0)),
        ],
        out_specs=[
            pl.BlockSpec((G, 1), lambda i: (0, 0)),
            pl.BlockSpec((TOPK, 1), lambda i: (0, 0)),
            pl.BlockSpec((TOPK, 1), lambda i: (0, 0)),
        ],
        out_shape=[
            jax.ShapeDtypeStruct((G, 1), jnp.float32),
            jax.ShapeDtypeStruct((TOPK, 1), jnp.float32),
            jax.ShapeDtypeStruct((TOPK, 1), jnp.int32),
        ],
        scratch_shapes=[
            pltpu.VMEM((G, 1), jnp.float32),
            pltpu.VMEM((G, 1), jnp.float32),
            pltpu.VMEM((G, D), jnp.float32),
            pltpu.VMEM((NBLK, BLK), jnp.float32),
            pltpu.VMEM((NBLK, BLK), jnp.int32),
            pltpu.VMEM((NBLK, BLK), jnp.float32),
        ],
        interpret=interpret,
    )(seg3, x, gwt, gb, W1, b1r, W2, b2r)

    return out, vals.reshape(TOPK), idx.reshape(TOPK)
